# baseline (speedup 1.0000x reference)
import functools

import numpy as np
import jax
import jax.numpy as jnp
from jax import lax
from jax.experimental import pallas as pl
from jax.experimental.pallas import tpu as pltpu
from jax.experimental.pallas import tpu_sc as plsc

_VOCAB = 1000000
_D = 64
_B = 1024
_S = 200
_T = _B * _S
_NELEM = _T * _D
_SCALE = np.float32(5.0 / np.sqrt(_S * _D))

_LANES = 128
_NROWS = _NELEM // _LANES
_BLK = 512
_GRID = _NROWS // _BLK


def _threefry_eps(p):
    ks0 = jnp.uint32(0)
    ks1 = jnp.uint32(1)
    ks2 = jnp.uint32(0x1BD11BDB)
    x0 = jnp.full_like(p, ks0)
    x1 = p + ks1
    rot0 = (13, 15, 26, 6)
    rot1 = (17, 29, 16, 24)
    schedule = (
        (rot0, ks1, ks2, 1),
        (rot1, ks2, ks0, 2),
        (rot0, ks0, ks1, 3),
        (rot1, ks1, ks2, 4),
        (rot0, ks2, ks0, 5),
    )
    for rots, ka, kb, c in schedule:
        for r in rots:
            x0 = x0 + x1
            x1 = (x1 << jnp.uint32(r)) | (x1 >> jnp.uint32(32 - r))
            x1 = x0 ^ x1
        x0 = x0 + ka
        x1 = x1 + kb + jnp.uint32(c)
    bits = x0 ^ x1
    fbits = (bits >> jnp.uint32(9)) | jnp.uint32(0x3F800000)
    return lax.bitcast_convert_type(fbits, jnp.float32) - jnp.float32(1.0)


def _noise_add_body(x_ref, o_ref):
    b = pl.program_id(0)
    base = b.astype(jnp.uint32) * jnp.uint32(_BLK * _LANES)
    i = lax.broadcasted_iota(jnp.uint32, (_BLK, _LANES), 0)
    j = lax.broadcasted_iota(jnp.uint32, (_BLK, _LANES), 1)
    p = base + i * jnp.uint32(_LANES) + j
    o_ref[...] = x_ref[...] + _SCALE * _threefry_eps(p)


def _noise_add(xs2d, interpret=False):
    return pl.pallas_call(
        _noise_add_body,
        grid=(_GRID,),
        in_specs=[pl.BlockSpec((_BLK, _LANES), lambda b: (b, 0))],
        out_specs=pl.BlockSpec((_BLK, _LANES), lambda b: (b, 0)),
        out_shape=jax.ShapeDtypeStruct((_NROWS, _LANES), jnp.float32),
        interpret=interpret,
    )(xs2d)


_NW = 32
_TPW = _T // _NW
_CHUNK = 256
_NCH = _TPW // _CHUNK
_CROWS = _CHUNK // 2
_NBUF = 4


def _sc_gather_body(idx_hbm, table_hbm, out_hbm, idx_v, bufs, *sems):
    gsems = sems[:_NBUF]
    osems = sems[_NBUF:]
    w = lax.axis_index("s") * 2 + lax.axis_index("c")
    row_base = w * (_TPW // 2)
    pltpu.sync_copy(idx_hbm.at[pl.ds(w * 2 * _NCH, 2 * _NCH)], idx_v)

    def start_gather(j):
        b = j % _NBUF
        return (
            pltpu.async_copy(table_hbm.at[idx_v.at[2 * j]], bufs.at[b, 0],
                             gsems[b]),
            pltpu.async_copy(table_hbm.at[idx_v.at[2 * j + 1]], bufs.at[b, 1],
                             gsems[b]),
        )

    gdesc = [None] * _NCH
    odesc = [None] * _NCH
    for j in range(min(2, _NCH)):
        gdesc[j] = start_gather(j)
    for j in range(_NCH):
        b = j % _NBUF
        gdesc[j][0].wait()
        gdesc[j][1].wait()
        r = row_base + j * _CROWS
        odesc[j] = (
            pltpu.async_copy(bufs.at[b, 0],
                             out_hbm.at[pl.ds(r, _CROWS), pl.ds(0, _D)],
                             osems[b]),
            pltpu.async_copy(bufs.at[b, 1],
                             out_hbm.at[pl.ds(r, _CROWS), pl.ds(_D, _D)],
                             osems[b]),
        )
        nj = j + 2
        if nj < _NCH:
            if nj - _NBUF >= 0:
                odesc[nj - _NBUF][0].wait()
                odesc[nj - _NBUF][1].wait()
            gdesc[nj] = start_gather(nj)
    for j in range(_NCH - _NBUF, _NCH):
        odesc[j][0].wait()
        odesc[j][1].wait()


def _sc_gather(ids_flat, table):
    mesh = plsc.VectorSubcoreMesh(core_axis_name="c", subcore_axis_name="s")
    scratch = [
        pltpu.VMEM((2 * _NCH, _CROWS), jnp.int32),
        pltpu.VMEM((_NBUF, 2, _CROWS, _D), jnp.float32),
    ] + [pltpu.SemaphoreType.DMA] * (2 * _NBUF)
    k = pl.kernel(
        _sc_gather_body,
        out_type=jax.ShapeDtypeStruct((_NROWS, _LANES), jnp.float32),
        mesh=mesh,
        scratch_types=scratch,
        compiler_params=pltpu.CompilerParams(use_tc_tiling_on_sc=False),
    )
    idx = (ids_flat.reshape(_NW, _NCH, _CROWS, 2)
           .transpose(0, 1, 3, 2)
           .reshape(_NW * _NCH * 2, _CROWS))
    return k(idx, table)


def kernel(input_ids, table):
    g2d = _sc_gather(input_ids.reshape(-1), table)
    out2d = _noise_add(g2d)
    return out2d.reshape(_B, _S, _D)


# speedup vs baseline: 1.0880x; 1.0880x over previous
import functools

import numpy as np
import jax
import jax.numpy as jnp
from jax import lax
from jax.experimental import pallas as pl
from jax.experimental.pallas import tpu as pltpu
from jax.experimental.pallas import tpu_sc as plsc

_VOCAB = 1000000
_D = 64
_B = 1024
_S = 200
_T = _B * _S
_NELEM = _T * _D
_SCALE = np.float32(5.0 / np.sqrt(_S * _D))

_LANES = 128
_NROWS = _NELEM // _LANES
_BLK = 512
_GRID = _NROWS // _BLK


def _threefry_eps(p):
    ks0 = jnp.uint32(0)
    ks1 = jnp.uint32(1)
    ks2 = jnp.uint32(0x1BD11BDB)
    x0 = jnp.full_like(p, ks0)
    x1 = p + ks1
    rot0 = (13, 15, 26, 6)
    rot1 = (17, 29, 16, 24)
    schedule = (
        (rot0, ks1, ks2, 1),
        (rot1, ks2, ks0, 2),
        (rot0, ks0, ks1, 3),
        (rot1, ks1, ks2, 4),
        (rot0, ks2, ks0, 5),
    )
    for rots, ka, kb, c in schedule:
        for r in rots:
            x0 = x0 + x1
            x1 = (x1 << jnp.uint32(r)) | (x1 >> jnp.uint32(32 - r))
            x1 = x0 ^ x1
        x0 = x0 + ka
        x1 = x1 + kb + jnp.uint32(c)
    bits = x0 ^ x1
    fbits = (bits >> jnp.uint32(9)) | jnp.uint32(0x3F800000)
    return lax.bitcast_convert_type(fbits, jnp.float32) - jnp.float32(1.0)


_BB = 32
_HTOK = _BB * _S // 2
_BROWS = _HTOK


def _noise_add_body(x_ref, o_ref):
    b = pl.program_id(0)
    base = b.astype(jnp.uint32) * jnp.uint32(2 * _HTOK * _D)
    i = lax.broadcasted_iota(jnp.uint32, (_BROWS, _LANES), 0)
    j = lax.broadcasted_iota(jnp.uint32, (_BROWS, _LANES), 1)
    p = base + i * jnp.uint32(_D) + j + jnp.where(
        j < _D, jnp.uint32(0), jnp.uint32(_HTOK * _D - _D))
    y = x_ref[...] + _SCALE * _threefry_eps(p)
    hb = _BB // 2
    o_ref[pl.ds(0, hb), :, :] = y[:, :_D].reshape(hb, _S, _D)
    o_ref[pl.ds(hb, hb), :, :] = y[:, _D:].reshape(hb, _S, _D)


def _noise_add(xs2d, interpret=False):
    return pl.pallas_call(
        _noise_add_body,
        grid=(_B // _BB,),
        in_specs=[pl.BlockSpec((_BROWS, _LANES), lambda b: (b, 0))],
        out_specs=pl.BlockSpec((_BB, _S, _D), lambda b: (b, 0, 0)),
        out_shape=jax.ShapeDtypeStruct((_B, _S, _D), jnp.float32),
        interpret=interpret,
    )(xs2d)


_NW = 32
_TPW = _T // _NW
_CHUNK = 128
_NCH = _TPW // _CHUNK
_HCH = _NCH // 2
_NBUF = 4


def _sc_gather_body(idx_hbm, table_hbm, out_hbm, idx_v, bufs, *sems):
    gsems = sems[:_NBUF]
    osems = sems[_NBUF:]
    w = lax.axis_index("s") * 2 + lax.axis_index("c")
    row_base = w * (_TPW // 2)
    pltpu.sync_copy(idx_hbm.at[pl.ds(w * _NCH, _NCH)], idx_v)

    def start_gather(j):
        b = j % _NBUF
        return pltpu.async_copy(table_hbm.at[idx_v.at[j]], bufs.at[b],
                                gsems[b])

    def start_out(j):
        b = j % _NBUF
        half, jj = divmod(j, _HCH)
        dst = out_hbm.at[pl.ds(row_base + jj * _CHUNK, _CHUNK),
                         pl.ds(half * _D, _D)]
        return pltpu.async_copy(bufs.at[b], dst, osems[b])

    gdesc = [None] * _NCH
    odesc = [None] * _NCH
    for j in range(min(2, _NCH)):
        gdesc[j] = start_gather(j)
    for j in range(_NCH):
        gdesc[j].wait()
        odesc[j] = start_out(j)
        nj = j + 2
        if nj < _NCH:
            if nj - _NBUF >= 0:
                odesc[nj - _NBUF].wait()
            gdesc[nj] = start_gather(nj)
    for j in range(_NCH - _NBUF, _NCH):
        odesc[j].wait()


def _sc_gather(ids_flat, table):
    mesh = plsc.VectorSubcoreMesh(core_axis_name="c", subcore_axis_name="s")
    scratch = [
        pltpu.VMEM((_NCH, _CHUNK), jnp.int32),
        pltpu.VMEM((_NBUF, _CHUNK, _D), jnp.float32),
    ] + [pltpu.SemaphoreType.DMA] * (2 * _NBUF)
    k = pl.kernel(
        _sc_gather_body,
        out_type=jax.ShapeDtypeStruct((_NROWS, _LANES), jnp.float32),
        mesh=mesh,
        scratch_types=scratch,
        compiler_params=pltpu.CompilerParams(use_tc_tiling_on_sc=False),
    )
    return k(ids_flat.reshape(_NW * _NCH, _CHUNK), table)


def kernel(input_ids, table):
    g2d = _sc_gather(input_ids.reshape(-1), table)
    return _noise_add(g2d)
